# Initial kernel scaffold; baseline (speedup 1.0000x reference)
import jax
import jax.numpy as jnp
from jax.experimental import pallas as pl


def kernel(pos, z, edge_index, W_embed, b_embed, Wsc, Wl1, Wfc1, Wfc2, Wl2, W_out):
    raise NotImplementedError("write your pallas kernel here")


# speedup vs baseline: 2.1546x; 2.1546x over previous
import functools
import math

import jax
import jax.numpy as jnp
from jax import lax
from jax.experimental import pallas as pl
from jax.experimental.pallas import tpu as pltpu
from jax.experimental.pallas import tpu_sc as plsc

N = 10000
E = 320000
D = 128
NB = 10
RH = 128
NCONV = 3
MAXR = 2.5
AVGDEG = 32.0
ZMAX = 118
OUT = 3

NC = 2
NS = 16
NW = NC * NS
EW = E // NW
CH = 80
NCHUNK = EW // CH
IBR = 5
NIB = NCHUNK // IBR
STRIPE = 624
LAST_BASE = STRIPE * (NS - 1)
LAST_STRIPE = N - LAST_BASE
NODE_W = 25
NPW = N // NODE_W

CS = math.sin(math.pi / 8)
CX = math.cos(math.pi / 8)
STEP = MAXR / (NB - 1)

_SC_MESH = plsc.VectorSubcoreMesh(
    core_axis_name="c", subcore_axis_name="s", num_cores=NC, num_subcores=NS)


def _k0_body(px_hbm, py_hbm, pz_hbm, src4_hbm, dst4_hbm, z_hbm, wemb_hbm,
             r2_hbm, x0_hbm,
             sidx, didx, px, py, pz, r2buf, zidx, xrows):
    c = lax.axis_index("c")
    s = lax.axis_index("s")
    w = c * NS + s

    pltpu.sync_copy(px_hbm, px)
    pltpu.sync_copy(py_hbm, py)
    pltpu.sync_copy(pz_hbm, pz)

    def slab(k, carry):
        pltpu.sync_copy(src4_hbm.at[w, k], sidx)
        pltpu.sync_copy(dst4_hbm.at[w, k], didx)
        for j in range(IBR):
            for g in range(CH // 16):
                sl = pl.ds(g * 16, 16)
                siv = sidx[j, sl]
                div = didx[j, sl]
                dx = plsc.load_gather(px, [siv]) - plsc.load_gather(px, [div])
                dy = plsc.load_gather(py, [siv]) - plsc.load_gather(py, [div])
                dz = plsc.load_gather(pz, [siv]) - plsc.load_gather(pz, [div])
                r2buf[pl.ds(j * CH + g * 16, 16)] = dx * dx + dy * dy + dz * dz
        pltpu.sync_copy(r2buf, r2_hbm.at[pl.ds(w * EW + k * IBR * CH, IBR * CH)])
        return carry

    lax.fori_loop(0, NIB, slab, 0)

    @pl.when(w < NODE_W)
    def _():
        pltpu.sync_copy(z_hbm.at[pl.ds(w * NPW, NPW)], zidx)
        pltpu.sync_copy(wemb_hbm.at[zidx], xrows)
        pltpu.sync_copy(xrows, x0_hbm.at[pl.ds(w * NPW, NPW)])


_k0 = pl.kernel(
    _k0_body,
    out_type=(jax.ShapeDtypeStruct((E,), jnp.float32),
              jax.ShapeDtypeStruct((N, D), jnp.float32)),
    mesh=_SC_MESH,
    compiler_params=pltpu.CompilerParams(needs_layout_passes=False),
    scratch_types=[
        pltpu.VMEM((IBR, CH), jnp.int32),
        pltpu.VMEM((IBR, CH), jnp.int32),
        pltpu.VMEM((N,), jnp.float32),
        pltpu.VMEM((N,), jnp.float32),
        pltpu.VMEM((N,), jnp.float32),
        pltpu.VMEM((IBR * CH,), jnp.float32),
        pltpu.VMEM((NPW,), jnp.int32),
        pltpu.VMEM((NPW, D), jnp.float32),
    ],
)


def _k2_body(x1_hbm, c_hbm, src4_hbm, dst4_hbm, zeros_hbm,
             part_hbm,
             sidx, didx, rows, cbuf, acc):
    c = lax.axis_index("c")
    s = lax.axis_index("s")
    w = c * NS + s

    @pl.when(s < NS - 1)
    def _():
        pltpu.sync_copy(zeros_hbm.at[pl.ds(s * STRIPE, STRIPE)],
                        acc.at[pl.ds(s * STRIPE, STRIPE)])

    @pl.when(s == NS - 1)
    def _():
        pltpu.sync_copy(zeros_hbm.at[pl.ds(LAST_BASE, LAST_STRIPE)],
                        acc.at[pl.ds(LAST_BASE, LAST_STRIPE)])

    plsc.subcore_barrier()

    def slab(k, carry):
        pltpu.sync_copy(src4_hbm.at[w, k], sidx)
        pltpu.sync_copy(dst4_hbm.at[w, k], didx)
        for j in range(IBR):
            pltpu.sync_copy(x1_hbm.at[sidx.at[j]], rows)
            pltpu.sync_copy(
                c_hbm.at[pl.ds(w * EW + (k * IBR + j) * CH, CH)], cbuf)

            def mul(e, carry2):
                for f in range(D // 16):
                    sl = pl.ds(f * 16, 16)
                    rows[e, sl] = rows[e, sl] * cbuf[e, sl]
                return carry2

            lax.fori_loop(0, CH, mul, 0)
            pltpu.sync_copy(rows, acc.at[didx.at[j]], add=True)
        return carry

    lax.fori_loop(0, NIB, slab, 0)
    plsc.subcore_barrier()

    @pl.when(s < NS - 1)
    def _():
        pltpu.sync_copy(acc.at[pl.ds(s * STRIPE, STRIPE)],
                        part_hbm.at[c, pl.ds(s * STRIPE, STRIPE)])

    @pl.when(s == NS - 1)
    def _():
        pltpu.sync_copy(acc.at[pl.ds(LAST_BASE, LAST_STRIPE)],
                        part_hbm.at[c, pl.ds(LAST_BASE, LAST_STRIPE)])


_k2 = pl.kernel(
    _k2_body,
    out_type=jax.ShapeDtypeStruct((NC, N, D), jnp.float32),
    mesh=_SC_MESH,
    compiler_params=pltpu.CompilerParams(needs_layout_passes=False),
    scratch_types=[
        pltpu.VMEM((IBR, CH), jnp.int32),
        pltpu.VMEM((IBR, CH), jnp.int32),
        pltpu.VMEM((CH, D), jnp.float32),
        pltpu.VMEM((CH, D), jnp.float32),
        pltpu.VMEM_SHARED((N, D), jnp.float32),
    ],
)


_NBLK = 400


def _a1_body(x_ref, b_ref, wsc_ref, wl1_ref, s_ref, x1_ref):
    xb = x_ref[...] + b_ref[...]
    s_ref[...] = jnp.dot(xb, wsc_ref[...],
                         preferred_element_type=jnp.float32) * (CS / math.sqrt(D))
    x1_ref[...] = jnp.dot(xb, wl1_ref[...],
                          preferred_element_type=jnp.float32) * (1.0 / math.sqrt(D))


def _a1(x, b, wsc, wl1):
    return pl.pallas_call(
        _a1_body,
        grid=(N // _NBLK,),
        in_specs=[
            pl.BlockSpec((_NBLK, D), lambda i: (i, 0)),
            pl.BlockSpec((1, D), lambda i: (0, 0)),
            pl.BlockSpec((D, D), lambda i: (0, 0)),
            pl.BlockSpec((D, D), lambda i: (0, 0)),
        ],
        out_specs=[
            pl.BlockSpec((_NBLK, D), lambda i: (i, 0)),
            pl.BlockSpec((_NBLK, D), lambda i: (i, 0)),
        ],
        out_shape=[jax.ShapeDtypeStruct((N, D), jnp.float32),
                   jax.ShapeDtypeStruct((N, D), jnp.float32)],
    )(x, b, wsc, wl1)


_EBLK = 1280


def _a2_body(r2_ref, wfc1_ref, wfc2_ref, c_ref):
    r2 = r2_ref[...]
    elen = jnp.sqrt(r2 + 1e-9)
    lane_i = lax.broadcasted_iota(jnp.int32, (_EBLK, 16), 1)
    lane = lane_i.astype(jnp.float32)
    diff = (elen - lane * STEP) * (1.0 / STEP)
    emb = jnp.where(lane_i < NB, jnp.exp(-(diff * diff)) * (1.0 / 1.12), 0.0)
    h = jnp.dot(emb, wfc1_ref[...], preferred_element_type=jnp.float32)
    a = h * jax.nn.sigmoid(h)
    u = 2.0 * (elen * (1.0 / MAXR) - 1.0)
    y = (1.0 - jnp.cos(math.pi * u)) * 0.5
    y = jnp.where(u > 0, 0.0, y)
    y = jnp.where(u < -1, 1.0, y)
    a = a * y
    c_ref[...] = jnp.dot(a, wfc2_ref[...],
                         preferred_element_type=jnp.float32) * (1.0 / math.sqrt(RH))


def _a2(r2c, wfc1p, wfc2):
    return pl.pallas_call(
        _a2_body,
        grid=(E // _EBLK,),
        in_specs=[
            pl.BlockSpec((_EBLK, 1), lambda i: (i, 0)),
            pl.BlockSpec((16, RH), lambda i: (0, 0)),
            pl.BlockSpec((RH, D), lambda i: (0, 0)),
        ],
        out_specs=pl.BlockSpec((_EBLK, D), lambda i: (i, 0)),
        out_shape=jax.ShapeDtypeStruct((E, D), jnp.float32),
    )(r2c, wfc1p, wfc2)


def _k3_body(p_ref, s_ref, w2_ref, wo_ref, o_ref, *, last):
    t = p_ref[0] + p_ref[1]
    t = jnp.dot(t, w2_ref[...], preferred_element_type=jnp.float32) * (
        CX / (math.sqrt(D) * math.sqrt(AVGDEG)))
    xn = s_ref[...] + t
    if last:
        o_ref[...] = jnp.dot(xn, wo_ref[...], preferred_element_type=jnp.float32)
    else:
        o_ref[...] = xn * jax.nn.sigmoid(xn)


def _k3(part, s, wl2, woutp, last):
    return pl.pallas_call(
        functools.partial(_k3_body, last=last),
        grid=(N // _NBLK,),
        in_specs=[
            pl.BlockSpec((NC, _NBLK, D), lambda i: (0, i, 0)),
            pl.BlockSpec((_NBLK, D), lambda i: (i, 0)),
            pl.BlockSpec((D, D), lambda i: (0, 0)),
            pl.BlockSpec((D, D), lambda i: (0, 0)),
        ],
        out_specs=pl.BlockSpec((_NBLK, D), lambda i: (i, 0)),
        out_shape=jax.ShapeDtypeStruct((N, D), jnp.float32),
    )(part, s, wl2, woutp)


def kernel(pos, z, edge_index, W_embed, b_embed, Wsc, Wl1, Wfc1, Wfc2, Wl2, W_out):
    posx, posy, posz = pos[:, 0], pos[:, 1], pos[:, 2]
    src4 = edge_index[0].astype(jnp.int32).reshape(NW, NIB, IBR, CH)
    dst4 = edge_index[1].astype(jnp.int32).reshape(NW, NIB, IBR, CH)
    z32 = z.astype(jnp.int32)
    wfc1p = jnp.pad(Wfc1, ((0, 0), (0, 16 - NB), (0, 0)))
    woutp = jnp.pad(W_out, ((0, 0), (0, D - OUT)))
    zeros = jnp.zeros((N, D), jnp.float32)
    bias0 = b_embed.reshape(1, D)
    bias_z = jnp.zeros((1, D), jnp.float32)

    r2, x0 = _k0(posx, posy, posz, src4, dst4, z32, W_embed)
    r2c = r2.reshape(E, 1)

    x = x0
    for l in range(NCONV):
        s, x1 = _a1(x, bias0 if l == 0 else bias_z, Wsc[l], Wl1[l])
        cvec = _a2(r2c, wfc1p[l], Wfc2[l])
        part = _k2(x1, cvec, src4, dst4, zeros)
        x = _k3(part, s, Wl2[l], woutp, last=(l == NCONV - 1))
    return x[:, :OUT]
